# baseline (speedup 1.0000x reference)
import functools

import jax
import jax.numpy as jnp
from jax.experimental import pallas as pl
from jax.experimental.pallas import tpu as pltpu

_N = 10000
_E = 320000
_H = 128
_CH = 3200
_C = _E // _CH
_CW = 400
_CR = _CH // _CW
_NEG = 0.2


def _dense_pre_k(x_ref, asrc_ref, adst_ref, linr_ref, s_ref, d_ref, xr_ref):
    xv = x_ref[...]
    s_ref[...] = jnp.dot(xv, asrc_ref[...], preferred_element_type=jnp.float32)
    d_ref[...] = jnp.dot(xv, adst_ref[...], preferred_element_type=jnp.float32)
    xr_ref[...] = jnp.dot(xv, linr_ref[...], preferred_element_type=jnp.float32)


def _p1_k(src_ref, dst_ref, s_ref, d_ref, alpha_ref, amax_ref):
    @pl.when(pl.program_id(0) == 0)
    def _():
        amax_ref[...] = jnp.full_like(amax_ref[...], -jnp.inf)

    def body(i, carry):
        r, c = jnp.divmod(i, _CW)
        sj = src_ref[r, c]
        di = dst_ref[r, c]
        a = (s_ref[pl.ds(di, 1), :]
             + d_ref[pl.ds(sj, 1), :])
        a = jnp.where(a > 0, a, _NEG * a)
        alpha_ref[pl.ds(i, 1), :] = a
        m = amax_ref[pl.ds(di, 1), :]
        amax_ref[pl.ds(di, 1), :] = jnp.maximum(m, a)
        return carry

    jax.lax.fori_loop(0, _CH, body, 0)


def _p2_k(dst_ref, alpha_ref, amax_ref, p_ref, denom_ref):
    @pl.when(pl.program_id(0) == 0)
    def _():
        denom_ref[...] = jnp.zeros_like(denom_ref[...])

    def body(i, carry):
        r, c = jnp.divmod(i, _CW)
        di = dst_ref[r, c]
        a = alpha_ref[pl.ds(i, 1), :]
        m = amax_ref[pl.ds(di, 1), :]
        p = jnp.exp(a - m)
        p_ref[pl.ds(i, 1), :] = p
        dn = denom_ref[pl.ds(di, 1), :]
        denom_ref[pl.ds(di, 1), :] = dn + p
        return carry

    jax.lax.fori_loop(0, _CH, body, 0)


def _p3_k(src_ref, dst_ref, p_ref, denom_ref, x_ref, agg_ref):
    @pl.when(pl.program_id(0) == 0)
    def _():
        agg_ref[...] = jnp.full_like(agg_ref[...], -jnp.inf)

    def body(i, carry):
        r, c = jnp.divmod(i, _CW)
        sj = src_ref[r, c]
        di = dst_ref[r, c]
        p = p_ref[pl.ds(i, 1), :]
        dn = denom_ref[pl.ds(di, 1), :]
        w = p / (dn + 1e-16)
        row = x_ref[pl.ds(sj, 1), :]
        cur = agg_ref[pl.ds(di, 1), :]
        agg_ref[pl.ds(di, 1), :] = jnp.maximum(cur, w * row)
        return carry

    jax.lax.fori_loop(0, _CH, body, 0)


def _post_k(agg_ref, linl_ref, bias_ref, xr_ref, h_ref):
    agg = agg_ref[...]
    agg = jnp.where(jnp.isfinite(agg), agg, 0.0)
    h = (jnp.dot(agg, linl_ref[...], preferred_element_type=jnp.float32)
         + bias_ref[...] + xr_ref[...])
    h_ref[...] = jnp.maximum(h, 0.0)


def _final_k(h_ref, w_ref, b_ref, out_ref):
    out_ref[...] = (jnp.dot(h_ref[...], w_ref[...],
                            preferred_element_type=jnp.float32) + b_ref[...])


def _full_spec(shape):
    return pl.BlockSpec(shape, lambda *_: tuple(0 for _ in shape))


def _layer(xv, src, dst, lin_l_w, lin_l_b, lin_r_w, att_src, att_dst):
    s, d, xr = pl.pallas_call(
        _dense_pre_k,
        out_shape=[
            jax.ShapeDtypeStruct((_N, 1), jnp.float32),
            jax.ShapeDtypeStruct((_N, 1), jnp.float32),
            jax.ShapeDtypeStruct((_N, _H), jnp.float32),
        ],
    )(xv, att_src, att_dst, lin_r_w)

    chunk_smem = pl.BlockSpec((_CR, _CW), lambda c: (c, 0),
                              memory_space=pltpu.SMEM)
    alpha, amax = pl.pallas_call(
        _p1_k,
        grid=(_C,),
        in_specs=[chunk_smem, chunk_smem,
                  pl.BlockSpec((_N, 1), lambda c: (0, 0)),
                  pl.BlockSpec((_N, 1), lambda c: (0, 0))],
        out_specs=[pl.BlockSpec((_CH, 1), lambda c: (c, 0)),
                   pl.BlockSpec((_N, 1), lambda c: (0, 0))],
        out_shape=[jax.ShapeDtypeStruct((_E, 1), jnp.float32),
                   jax.ShapeDtypeStruct((_N, 1), jnp.float32)],
    )(src, dst, s, d)

    p, denom = pl.pallas_call(
        _p2_k,
        grid=(_C,),
        in_specs=[chunk_smem,
                  pl.BlockSpec((_CH, 1), lambda c: (c, 0)),
                  pl.BlockSpec((_N, 1), lambda c: (0, 0))],
        out_specs=[pl.BlockSpec((_CH, 1), lambda c: (c, 0)),
                   pl.BlockSpec((_N, 1), lambda c: (0, 0))],
        out_shape=[jax.ShapeDtypeStruct((_E, 1), jnp.float32),
                   jax.ShapeDtypeStruct((_N, 1), jnp.float32)],
    )(dst, alpha, amax)

    agg = pl.pallas_call(
        _p3_k,
        grid=(_C,),
        in_specs=[chunk_smem, chunk_smem,
                  pl.BlockSpec((_CH, 1), lambda c: (c, 0)),
                  pl.BlockSpec((_N, 1), lambda c: (0, 0)),
                  pl.BlockSpec((_N, _H), lambda c: (0, 0))],
        out_specs=pl.BlockSpec((_N, _H), lambda c: (0, 0)),
        out_shape=jax.ShapeDtypeStruct((_N, _H), jnp.float32),
    )(src, dst, p, denom, xv)

    h = pl.pallas_call(
        _post_k,
        out_shape=jax.ShapeDtypeStruct((_N, _H), jnp.float32),
    )(agg, lin_l_w, lin_l_b.reshape(1, _H), xr)
    return h


def kernel(x, edge_index, lin_l0_w, lin_l0_b, lin_r0_w, att_src0, att_dst0,
           lin_l1_w, lin_l1_b, lin_r1_w, att_src1, att_dst1, out_w, out_b):
    src = edge_index[0].reshape(_C * _CR, _CW)
    dst = edge_index[1].reshape(_C * _CR, _CW)
    h = _layer(x, src, dst, lin_l0_w, lin_l0_b, lin_r0_w, att_src0, att_dst0)
    h = _layer(h, src, dst, lin_l1_w, lin_l1_b, lin_r1_w, att_src1, att_dst1)
    out = pl.pallas_call(
        _final_k,
        out_shape=jax.ShapeDtypeStruct((_N, 1), jnp.float32),
    )(h, out_w, out_b.reshape(1, 1))
    return out
